# baseline (speedup 1.0000x reference)
import functools

import jax
import jax.numpy as jnp
from jax import lax
from jax.experimental import pallas as pl
from jax.experimental.pallas import tpu as pltpu
from jax.experimental.pallas import tpu_sc as plsc

V = 1000000
D = 16
B = 16384
L = 50
N = B * L

NC = 2
NS = 16
NW = NC * NS
PER_W = N // NW
C = 2560
NCH = PER_W // C


def _dequant_lookup(idx_hbm, w_hbm, scale_hbm, out_hbm,
                    idx_v, idx2_v, rows_v, scale_v, out_v, sem):
    wid = lax.axis_index("s") * NC + lax.axis_index("c")
    wbase = wid * PER_W

    iota = lax.iota(jnp.int32, 16)
    qiota = iota >> 2
    riota = iota & 3
    siota = iota * 4

    def halve(m, carry):
        val = idx_v[pl.ds(m * 16, 16)]
        idx2_v[pl.ds(m * 16, 16)] = val >> 1
        return carry

    def body(k, carry):
        look = qiota + 4 * k
        par = plsc.load_gather(idx_v, [look]) & 1
        v = plsc.load_gather(rows_v, [look, riota + 4 * par])
        s = plsc.load_gather(scale_v, [look])
        base = k * 64
        b0 = ((v << 24) >> 24).astype(jnp.float32) * s
        b1 = ((v << 16) >> 24).astype(jnp.float32) * s
        b2 = ((v << 8) >> 24).astype(jnp.float32) * s
        b3 = (v >> 24).astype(jnp.float32) * s
        plsc.store_scatter(out_v, [siota + base], b0)
        plsc.store_scatter(out_v, [siota + (base + 1)], b1)
        plsc.store_scatter(out_v, [siota + (base + 2)], b2)
        plsc.store_scatter(out_v, [siota + (base + 3)], b3)
        return carry

    for ch in range(NCH):
        base = wbase + ch * C
        pltpu.sync_copy(idx_hbm.at[pl.ds(base, C)], idx_v)
        lax.fori_loop(0, C // 16, halve, 0, unroll=2)
        rows_dma = pltpu.async_copy(w_hbm.at[idx2_v], rows_v, sem)
        scale_dma = pltpu.async_copy(scale_hbm.at[idx_v], scale_v, sem)
        rows_dma.wait()
        scale_dma.wait()
        lax.fori_loop(0, C * D // 64, body, 0, unroll=2)
        pltpu.sync_copy(out_v, out_hbm.at[pl.ds(base * D, C * D)])


@jax.jit
def _run(idx, w8, scale):
    mesh = plsc.VectorSubcoreMesh(core_axis_name="c", subcore_axis_name="s")
    f = functools.partial(
        pl.kernel,
        mesh=mesh,
        out_type=jax.ShapeDtypeStruct((N * D,), jnp.float32),
        scratch_types=[
            pltpu.VMEM((C,), jnp.int32),
            pltpu.VMEM((C,), jnp.int32),
            pltpu.VMEM((C, 8), jnp.int32),
            pltpu.VMEM((C,), jnp.float32),
            pltpu.VMEM((C * D,), jnp.float32),
            pltpu.SemaphoreType.DMA,
        ],
        compiler_params=pltpu.CompilerParams(
            needs_layout_passes=False, use_tc_tiling_on_sc=False),
    )(_dequant_lookup)
    return f(idx, w8, scale)


def kernel(input, weight, weight_scale):
    idx = input.reshape(-1)
    w8 = lax.bitcast_convert_type(weight.reshape(V // 2, 8, 4), jnp.int32)
    out = _run(idx, w8, weight_scale)
    return out.reshape(B, L, D)


# speedup vs baseline: 1.6623x; 1.6623x over previous
import functools

import jax
import jax.numpy as jnp
from jax import lax
from jax.experimental import pallas as pl
from jax.experimental.pallas import tpu as pltpu
from jax.experimental.pallas import tpu_sc as plsc

V = 1000000
D = 16
B = 16384
L = 50
N = B * L

NC = 2
NS = 16
NW = NC * NS
PER_W = N // NW
C = 1600
NCH = PER_W // C


def _dequant_lookup(idx_hbm, w_hbm, scale_hbm, out_hbm,
                    idx_v, idxq_v, idxs_v, rows_v, scale_v, out_v, sem):
    wid = lax.axis_index("s") * NC + lax.axis_index("c")
    wbase = wid * PER_W

    iota = lax.iota(jnp.int32, 16)
    qiota = iota >> 2
    riota = iota & 3
    siota = iota * 4

    def recidx(m, carry):
        val = idx_v[pl.ds(m * 16, 16)]
        idxq_v[pl.ds(m * 16, 16)] = val >> 2
        idxs_v[pl.ds(m * 16, 16)] = val >> 4
        return carry

    def body(k, carry):
        look = qiota + 4 * k
        ql = plsc.load_gather(idx_v, [look])
        sub = ((ql & 3) << 2) + riota
        v = plsc.load_gather(rows_v, [look, sub])
        s = plsc.load_gather(scale_v, [look, ql & 15])
        base = k * 64
        b0 = ((v << 24) >> 24).astype(jnp.float32) * s
        b1 = ((v << 16) >> 24).astype(jnp.float32) * s
        b2 = ((v << 8) >> 24).astype(jnp.float32) * s
        b3 = (v >> 24).astype(jnp.float32) * s
        plsc.store_scatter(out_v, [siota + base], b0)
        plsc.store_scatter(out_v, [siota + (base + 1)], b1)
        plsc.store_scatter(out_v, [siota + (base + 2)], b2)
        plsc.store_scatter(out_v, [siota + (base + 3)], b3)
        return carry

    for ch in range(NCH):
        base = wbase + ch * C
        pltpu.sync_copy(idx_hbm.at[pl.ds(base, C)], idx_v)
        lax.fori_loop(0, C // 16, recidx, 0, unroll=2)
        rows_dma = pltpu.async_copy(w_hbm.at[idxq_v], rows_v, sem)
        scale_dma = pltpu.async_copy(scale_hbm.at[idxs_v], scale_v, sem)
        rows_dma.wait()
        scale_dma.wait()
        lax.fori_loop(0, C * D // 64, body, 0, unroll=2)
        pltpu.sync_copy(out_v, out_hbm.at[pl.ds(base * D, C * D)])


@jax.jit
def _run(idx, wq, scaleq):
    mesh = plsc.VectorSubcoreMesh(core_axis_name="c", subcore_axis_name="s")
    f = functools.partial(
        pl.kernel,
        mesh=mesh,
        out_type=jax.ShapeDtypeStruct((N * D,), jnp.float32),
        scratch_types=[
            pltpu.VMEM((C,), jnp.int32),
            pltpu.VMEM((C,), jnp.int32),
            pltpu.VMEM((C,), jnp.int32),
            pltpu.VMEM((C, 16), jnp.int32),
            pltpu.VMEM((C, 16), jnp.float32),
            pltpu.VMEM((C * D,), jnp.float32),
            pltpu.SemaphoreType.DMA,
        ],
        compiler_params=pltpu.CompilerParams(
            needs_layout_passes=False, use_tc_tiling_on_sc=False),
    )(_dequant_lookup)
    return f(idx, wq, scaleq)


def kernel(input, weight, weight_scale):
    idx = input.reshape(-1)
    wq = lax.bitcast_convert_type(weight.reshape(V // 4, 16, 4), jnp.int32)
    scaleq = weight_scale.reshape(V // 16, 16)
    out = _run(idx, wq, scaleq)
    return out.reshape(B, L, D)
